# Initial kernel scaffold; baseline (speedup 1.0000x reference)
import jax
import jax.numpy as jnp
from jax.experimental import pallas as pl


def kernel(inputs, task_indices, W1, b1, W2, b2, W3, b3):
    raise NotImplementedError("write your pallas kernel here")


# speedup vs baseline: 16.5675x; 16.5675x over previous
import functools

import jax
import jax.numpy as jnp
from jax.experimental import pallas as pl
from jax.experimental.pallas import tpu as pltpu

_BLOCK_ROWS = 1024


def _mlp_kernel(x_ref, w1_ref, b1_ref, w2_ref, b2_ref, w3_ref, b3_ref, out_ref):
    x = x_ref[...]
    h = jnp.tanh(
        jnp.dot(x, w1_ref[...], preferred_element_type=jnp.float32) + b1_ref[...]
    )
    h = jnp.tanh(
        jnp.dot(h, w2_ref[...], preferred_element_type=jnp.float32) + b2_ref[...]
    )
    out_ref[...] = (
        jnp.dot(h, w3_ref[...], preferred_element_type=jnp.float32) + b3_ref[...]
    )


@functools.partial(jax.jit, static_argnames=())
def _run(inputs, W1, b1, W2, b2, W3, b3):
    n, k = inputs.shape
    h = W1.shape[1]
    o = W3.shape[1]
    grid = (n // _BLOCK_ROWS,)
    const_spec = lambda shape: pl.BlockSpec(shape, lambda i: (0, 0))
    return pl.pallas_call(
        _mlp_kernel,
        grid=grid,
        in_specs=[
            pl.BlockSpec((_BLOCK_ROWS, k), lambda i: (i, 0)),
            const_spec((k, h)),
            const_spec((1, h)),
            const_spec((h, h)),
            const_spec((1, h)),
            const_spec((h, o)),
            const_spec((1, o)),
        ],
        out_specs=pl.BlockSpec((_BLOCK_ROWS, o), lambda i: (i, 0)),
        out_shape=jax.ShapeDtypeStruct((n, o), jnp.float32),
        compiler_params=pltpu.CompilerParams(
            dimension_semantics=("arbitrary",),
        ),
    )(inputs, W1, b1.reshape(1, h), W2, b2.reshape(1, h), W3, b3.reshape(1, o))


def kernel(inputs, task_indices, W1, b1, W2, b2, W3, b3):
    del task_indices
    return _run(inputs, W1, b1, W2, b2, W3, b3)


# speedup vs baseline: 19.2379x; 1.1612x over previous
import functools

import jax
import jax.numpy as jnp
from jax.experimental import pallas as pl
from jax.experimental.pallas import tpu as pltpu

_BLOCK_ROWS = 2048


def _mlp_kernel(x_ref, w1_ref, b1_ref, w2_ref, b2_ref, w3_ref, b3_ref, out_ref):
    x = x_ref[...]
    h = jnp.tanh(
        jnp.dot(x, w1_ref[...], preferred_element_type=jnp.float32) + b1_ref[...]
    )
    h = jnp.tanh(
        jnp.dot(h, w2_ref[...], preferred_element_type=jnp.float32) + b2_ref[...]
    )
    out_ref[...] = (
        jnp.dot(h, w3_ref[...], preferred_element_type=jnp.float32) + b3_ref[...]
    )


@functools.partial(jax.jit, static_argnames=())
def _run(inputs, W1, b1, W2, b2, W3, b3):
    n, k = inputs.shape
    h = W1.shape[1]
    o = W3.shape[1]
    grid = (n // _BLOCK_ROWS,)
    const_spec = lambda shape: pl.BlockSpec(shape, lambda i: (0, 0))
    return pl.pallas_call(
        _mlp_kernel,
        grid=grid,
        in_specs=[
            pl.BlockSpec((_BLOCK_ROWS, k), lambda i: (i, 0)),
            const_spec((k, h)),
            const_spec((1, h)),
            const_spec((h, h)),
            const_spec((1, h)),
            const_spec((h, o)),
            const_spec((1, o)),
        ],
        out_specs=pl.BlockSpec((_BLOCK_ROWS, o), lambda i: (i, 0)),
        out_shape=jax.ShapeDtypeStruct((n, o), jnp.float32),
        compiler_params=pltpu.CompilerParams(
            dimension_semantics=("arbitrary",),
        ),
    )(inputs, W1, b1.reshape(1, h), W2, b2.reshape(1, h), W3, b3.reshape(1, o))


def kernel(inputs, task_indices, W1, b1, W2, b2, W3, b3):
    del task_indices
    return _run(inputs, W1, b1, W2, b2, W3, b3)


# speedup vs baseline: 19.7175x; 1.0249x over previous
import functools

import jax
import jax.numpy as jnp
from jax.experimental import pallas as pl
from jax.experimental.pallas import tpu as pltpu

_BLOCK_ROWS = 4096


def _mlp_kernel(x_ref, w1_ref, b1_ref, w2_ref, b2_ref, w3_ref, b3_ref, out_ref):
    x = x_ref[...]
    h = jnp.tanh(
        jnp.dot(x, w1_ref[...], preferred_element_type=jnp.float32) + b1_ref[...]
    )
    h = jnp.tanh(
        jnp.dot(h, w2_ref[...], preferred_element_type=jnp.float32) + b2_ref[...]
    )
    out_ref[...] = (
        jnp.dot(h, w3_ref[...], preferred_element_type=jnp.float32) + b3_ref[...]
    )


@functools.partial(jax.jit, static_argnames=())
def _run(inputs, W1, b1, W2, b2, W3, b3):
    n, k = inputs.shape
    h = W1.shape[1]
    o = W3.shape[1]
    grid = (n // _BLOCK_ROWS,)
    const_spec = lambda shape: pl.BlockSpec(shape, lambda i: (0, 0))
    return pl.pallas_call(
        _mlp_kernel,
        grid=grid,
        in_specs=[
            pl.BlockSpec((_BLOCK_ROWS, k), lambda i: (i, 0)),
            const_spec((k, h)),
            const_spec((1, h)),
            const_spec((h, h)),
            const_spec((1, h)),
            const_spec((h, o)),
            const_spec((1, o)),
        ],
        out_specs=pl.BlockSpec((_BLOCK_ROWS, o), lambda i: (i, 0)),
        out_shape=jax.ShapeDtypeStruct((n, o), jnp.float32),
        compiler_params=pltpu.CompilerParams(
            dimension_semantics=("arbitrary",),
        ),
    )(inputs, W1, b1.reshape(1, h), W2, b2.reshape(1, h), W3, b3.reshape(1, o))


def kernel(inputs, task_indices, W1, b1, W2, b2, W3, b3):
    del task_indices
    return _run(inputs, W1, b1, W2, b2, W3, b3)


# speedup vs baseline: 19.8721x; 1.0078x over previous
import functools

import jax
import jax.numpy as jnp
from jax.experimental import pallas as pl
from jax.experimental.pallas import tpu as pltpu

_BLOCK_ROWS = 4096


def _mlp_kernel(
    xa_ref, xb_ref, w1a_ref, w1b_ref, b1_ref, w2_ref, b2_ref, w3_ref, b3_ref, out_ref
):
    h = jnp.tanh(
        jnp.dot(xa_ref[...], w1a_ref[...], preferred_element_type=jnp.float32)
        + jnp.dot(xb_ref[...], w1b_ref[...], preferred_element_type=jnp.float32)
        + b1_ref[...]
    )
    h = jnp.tanh(
        jnp.dot(h, w2_ref[...], preferred_element_type=jnp.float32) + b2_ref[...]
    )
    out_ref[...] = (
        jnp.dot(h, w3_ref[...], preferred_element_type=jnp.float32) + b3_ref[...]
    )


@functools.partial(jax.jit, static_argnames=())
def _run(inputs, W1, b1, W2, b2, W3, b3):
    n, k = inputs.shape
    h = W1.shape[1]
    o = W3.shape[1]
    kh = k // 2
    grid = (n // _BLOCK_ROWS,)
    const_spec = lambda shape: pl.BlockSpec(shape, lambda i: (0, 0))
    return pl.pallas_call(
        _mlp_kernel,
        grid=grid,
        in_specs=[
            pl.BlockSpec((_BLOCK_ROWS, kh), lambda i: (i, 0)),
            pl.BlockSpec((_BLOCK_ROWS, kh), lambda i: (i, 1)),
            const_spec((kh, h)),
            const_spec((kh, h)),
            const_spec((1, h)),
            const_spec((h, h)),
            const_spec((1, h)),
            const_spec((h, o)),
            const_spec((1, o)),
        ],
        out_specs=pl.BlockSpec((_BLOCK_ROWS, o), lambda i: (i, 0)),
        out_shape=jax.ShapeDtypeStruct((n, o), jnp.float32),
        compiler_params=pltpu.CompilerParams(
            dimension_semantics=("arbitrary",),
        ),
    )(
        inputs,
        inputs,
        W1[:kh],
        W1[kh:],
        b1.reshape(1, h),
        W2,
        b2.reshape(1, h),
        W3,
        b3.reshape(1, o),
    )


def kernel(inputs, task_indices, W1, b1, W2, b2, W3, b3):
    del task_indices
    return _run(inputs, W1, b1, W2, b2, W3, b3)


# speedup vs baseline: 19.8943x; 1.0011x over previous
import jax
import jax.numpy as jnp
from jax.experimental import pallas as pl
from jax.experimental.pallas import tpu as pltpu

_BLOCK_ROWS = 4096


def _mlp_kernel(x_ref, w1_ref, b1_ref, w2_ref, b2_ref, w3_ref, b3_ref, out_ref):
    h = jnp.tanh(
        jnp.dot(x_ref[...], w1_ref[...], preferred_element_type=jnp.float32)
        + b1_ref[...]
    )
    h = jnp.tanh(
        jnp.dot(h, w2_ref[...], preferred_element_type=jnp.float32) + b2_ref[...]
    )
    out_ref[...] = (
        jnp.dot(h, w3_ref[...], preferred_element_type=jnp.float32) + b3_ref[...]
    )


@jax.jit
def _run(inputs, W1, b1, W2, b2, W3, b3):
    n, k = inputs.shape
    h = W1.shape[1]
    o = W3.shape[1]
    const_spec = lambda shape: pl.BlockSpec(shape, lambda i: (0, 0))
    return pl.pallas_call(
        _mlp_kernel,
        grid=(n // _BLOCK_ROWS,),
        in_specs=[
            pl.BlockSpec((_BLOCK_ROWS, k), lambda i: (i, 0)),
            const_spec((k, h)),
            const_spec((1, h)),
            const_spec((h, h)),
            const_spec((1, h)),
            const_spec((h, o)),
            const_spec((1, o)),
        ],
        out_specs=pl.BlockSpec((_BLOCK_ROWS, o), lambda i: (i, 0)),
        out_shape=jax.ShapeDtypeStruct((n, o), jnp.float32),
        compiler_params=pltpu.CompilerParams(
            dimension_semantics=("arbitrary",),
        ),
    )(inputs, W1, b1.reshape(1, h), W2, b2.reshape(1, h), W3, b3.reshape(1, o))


def kernel(inputs, task_indices, W1, b1, W2, b2, W3, b3):
    del task_indices
    return _run(inputs, W1, b1, W2, b2, W3, b3)
